# baseline (speedup 1.0000x reference)
import jax
import jax.numpy as jnp
from jax.experimental import pallas as pl

_N = 10000
_INFEAT = 128
_HIDDEN = 24
_OUTFEAT = 128
_BM = 400


def _support_body(x_ref, w1_ref, o_ref):
    o_ref[...] = jnp.dot(x_ref[...], w1_ref[...],
                         preferred_element_type=jnp.float32)


def _main_body(adj_ref, s_ref, b1_ref, w2_ref, b2_ref, o_ref):
    acc = jnp.dot(adj_ref[...], s_ref[...],
                  preferred_element_type=jnp.float32)
    h = acc + b1_ref[...]
    h = jnp.where(h > 0, h, 0.01 * h)
    o_ref[...] = jnp.tanh(
        jnp.dot(h, w2_ref[...], preferred_element_type=jnp.float32)
        + b2_ref[...])


def kernel(x, adj, W1, b1, W2, b2):
    support = pl.pallas_call(
        _support_body,
        out_shape=jax.ShapeDtypeStruct((_N, _HIDDEN), jnp.float32),
    )(x, W1)

    b1r = b1.reshape(1, _HIDDEN)
    b2r = b2.reshape(1, _OUTFEAT)

    grid = (_N // _BM,)
    out = pl.pallas_call(
        _main_body,
        grid=grid,
        in_specs=[
            pl.BlockSpec((_BM, _N), lambda i: (i, 0)),
            pl.BlockSpec((_N, _HIDDEN), lambda i: (0, 0)),
            pl.BlockSpec((1, _HIDDEN), lambda i: (0, 0)),
            pl.BlockSpec((_HIDDEN, _OUTFEAT), lambda i: (0, 0)),
            pl.BlockSpec((1, _OUTFEAT), lambda i: (0, 0)),
        ],
        out_specs=pl.BlockSpec((_BM, _OUTFEAT), lambda i: (i, 0)),
        out_shape=jax.ShapeDtypeStruct((_N, _OUTFEAT), jnp.float32),
    )(adj, support, b1r, W2, b2r)
    return out


# speedup vs baseline: 1.0555x; 1.0555x over previous
import jax
import jax.numpy as jnp
from jax.experimental import pallas as pl
from jax.experimental.pallas import tpu as pltpu

_N = 10000
_INFEAT = 128
_HIDDEN = 24
_OUTFEAT = 128
_BM = 400


def _body(x_ref, adj_ref, w1_ref, b1_ref, w2_ref, b2_ref, o_ref, s_ref):
    @pl.when(pl.program_id(0) == 0)
    def _():
        s_ref[...] = jnp.dot(x_ref[...], w1_ref[...],
                             preferred_element_type=jnp.float32)

    acc = jnp.dot(adj_ref[...], s_ref[...],
                  preferred_element_type=jnp.float32)
    h = acc + b1_ref[...]
    h = jnp.where(h > 0, h, 0.01 * h)
    o_ref[...] = jnp.tanh(
        jnp.dot(h, w2_ref[...], preferred_element_type=jnp.float32)
        + b2_ref[...])


def kernel(x, adj, W1, b1, W2, b2):
    b1r = b1.reshape(1, _HIDDEN)
    b2r = b2.reshape(1, _OUTFEAT)

    return pl.pallas_call(
        _body,
        grid=(_N // _BM,),
        in_specs=[
            pl.BlockSpec((_N, _INFEAT), lambda i: (0, 0)),
            pl.BlockSpec((_BM, _N), lambda i: (i, 0)),
            pl.BlockSpec((_INFEAT, _HIDDEN), lambda i: (0, 0)),
            pl.BlockSpec((1, _HIDDEN), lambda i: (0, 0)),
            pl.BlockSpec((_HIDDEN, _OUTFEAT), lambda i: (0, 0)),
            pl.BlockSpec((1, _OUTFEAT), lambda i: (0, 0)),
        ],
        out_specs=pl.BlockSpec((_BM, _OUTFEAT), lambda i: (i, 0)),
        out_shape=jax.ShapeDtypeStruct((_N, _OUTFEAT), jnp.float32),
        scratch_shapes=[pltpu.VMEM((_N, _HIDDEN), jnp.float32)],
    )(x, adj, W1, b1r, W2, b2r)
